# baseline (speedup 1.0000x reference)
import jax
import jax.numpy as jnp
from jax import lax
from jax.experimental import pallas as pl
from jax.experimental.pallas import tpu as pltpu
from jax.experimental.pallas import tpu_sc as plsc

NUM_ACTIONS = 100000
ACTION_DIM = 32
BATCH = 16384
HIST = 50

NC, NS = 2, 16
NW = NC * NS
B_TOTAL = BATCH * HIST
PER_W = B_TOTAL // NW
CHUNK = 128
K = PER_W // CHUNK
SUB = 8
MEGA = CHUNK * SUB
N_MEGA = K // SUB
NBUF = 3


def _gather_body(idx_hbm, table_hbm, out_hbm, idx_v,
                 rows0, rows1, rows2, g0, g1, g2, o0, o1, o2, i0, i1):
    cid = lax.axis_index("c")
    sid = lax.axis_index("s")
    wid = sid * NC + cid
    base = wid * PER_W

    bufs = (rows0, rows1, rows2)
    gsems = (g0, g1, g2)
    osems = (o0, o1, o2)

    K2 = K // 2
    dh0 = pltpu.async_copy(idx_hbm.at[2 * wid], idx_v.at[pl.ds(0, K2)], i0)
    dh1 = pltpu.async_copy(idx_hbm.at[2 * wid + 1], idx_v.at[pl.ds(K2, K2)], i1)
    dh0.wait()

    def issue(m, b):
        for j in range(SUB):
            pltpu.async_copy(
                table_hbm.at[idx_v.at[m * SUB + j]],
                bufs[b].at[pl.ds(j * CHUNK, CHUNK)],
                gsems[b],
            )

    def drain(b):
        pltpu.make_async_copy(out_hbm.at[pl.ds(0, MEGA)], bufs[b], gsems[b]).wait()

    def out(m, b):
        pltpu.async_copy(bufs[b], out_hbm.at[pl.ds(base + m * MEGA, MEGA)], osems[b])

    def wait_out(b):
        pltpu.make_async_copy(bufs[b], out_hbm.at[pl.ds(0, MEGA)], osems[b]).wait()

    issue(0, 0)
    issue(1, 1)
    dh1.wait()

    drain(0)
    out(0, 0)
    issue(2, 2)

    @pl.loop(0, 7)
    def _steady(g):
        for b3 in range(3):
            m = 1 + 3 * g + b3
            db = (1 + b3) % 3
            nb = b3 % 3
            drain(db)
            out(m, db)
            wait_out(nb)
            issue(m + 2, nb)

    drain(1)
    out(22, 1)
    wait_out(0)
    issue(24, 0)
    drain(2)
    out(23, 2)
    drain(0)
    out(24, 0)

    wait_out(1)
    wait_out(2)
    wait_out(0)


_gather = pl.kernel(
    _gather_body,
    out_type=jax.ShapeDtypeStruct((B_TOTAL, ACTION_DIM), jnp.float32),
    mesh=plsc.VectorSubcoreMesh(core_axis_name="c", subcore_axis_name="s"),
    scratch_types=[
        pltpu.VMEM((K, CHUNK), jnp.int32),
        pltpu.VMEM((MEGA, ACTION_DIM), jnp.float32),
        pltpu.VMEM((MEGA, ACTION_DIM), jnp.float32),
        pltpu.VMEM((MEGA, ACTION_DIM), jnp.float32),
        pltpu.SemaphoreType.DMA,
        pltpu.SemaphoreType.DMA,
        pltpu.SemaphoreType.DMA,
        pltpu.SemaphoreType.DMA,
        pltpu.SemaphoreType.DMA,
        pltpu.SemaphoreType.DMA,
        pltpu.SemaphoreType.DMA,
        pltpu.SemaphoreType.DMA,
    ],
    compiler_params=pltpu.CompilerParams(use_tc_tiling_on_sc=False),
)


def kernel(action, table):
    idx = action.reshape(NW * 2, K // 2, CHUNK).astype(jnp.int32)
    out = _gather(idx, table)
    return out.reshape(BATCH, HIST * ACTION_DIM)


# speedup vs baseline: 1.0039x; 1.0039x over previous
import jax
import jax.numpy as jnp
from jax import lax
from jax.experimental import pallas as pl
from jax.experimental.pallas import tpu as pltpu
from jax.experimental.pallas import tpu_sc as plsc

NUM_ACTIONS = 100000
ACTION_DIM = 32
BATCH = 16384
HIST = 50

NC, NS = 2, 16
NW = NC * NS
B_TOTAL = BATCH * HIST
PER_W = B_TOTAL // NW
CHUNK = 256
K = PER_W // CHUNK
SUB = 4
MEGA = CHUNK * SUB
N_MEGA = K // SUB
NBUF = 3


def _gather_body(idx_hbm, table_hbm, out_hbm, idx_v,
                 rows0, rows1, rows2, g0, g1, g2, o0, o1, o2, i0, i1):
    cid = lax.axis_index("c")
    sid = lax.axis_index("s")
    wid = sid * NC + cid
    base = wid * PER_W

    bufs = (rows0, rows1, rows2)
    gsems = (g0, g1, g2)
    osems = (o0, o1, o2)

    K2 = K // 2
    dh0 = pltpu.async_copy(idx_hbm.at[2 * wid], idx_v.at[pl.ds(0, K2)], i0)
    dh1 = pltpu.async_copy(idx_hbm.at[2 * wid + 1], idx_v.at[pl.ds(K2, K2)], i1)
    dh0.wait()

    def issue(m, b):
        for j in range(SUB):
            pltpu.async_copy(
                table_hbm.at[idx_v.at[m * SUB + j]],
                bufs[b].at[pl.ds(j * CHUNK, CHUNK)],
                gsems[b],
            )

    def drain(b):
        pltpu.make_async_copy(out_hbm.at[pl.ds(0, MEGA)], bufs[b], gsems[b]).wait()

    def out(m, b):
        pltpu.async_copy(bufs[b], out_hbm.at[pl.ds(base + m * MEGA, MEGA)], osems[b])

    def wait_out(b):
        pltpu.make_async_copy(bufs[b], out_hbm.at[pl.ds(0, MEGA)], osems[b]).wait()

    issue(0, 0)
    issue(1, 1)
    dh1.wait()

    drain(0)
    out(0, 0)
    issue(2, 2)

    @pl.loop(0, 7)
    def _steady(g):
        for b3 in range(3):
            m = 1 + 3 * g + b3
            db = (1 + b3) % 3
            nb = b3 % 3
            drain(db)
            out(m, db)
            wait_out(nb)
            issue(m + 2, nb)

    drain(1)
    out(22, 1)
    wait_out(0)
    issue(24, 0)
    drain(2)
    out(23, 2)
    drain(0)
    out(24, 0)

    wait_out(1)
    wait_out(2)
    wait_out(0)


_gather = pl.kernel(
    _gather_body,
    out_type=jax.ShapeDtypeStruct((B_TOTAL, ACTION_DIM), jnp.float32),
    mesh=plsc.VectorSubcoreMesh(core_axis_name="c", subcore_axis_name="s"),
    scratch_types=[
        pltpu.VMEM((K, CHUNK), jnp.int32),
        pltpu.VMEM((MEGA, ACTION_DIM), jnp.float32),
        pltpu.VMEM((MEGA, ACTION_DIM), jnp.float32),
        pltpu.VMEM((MEGA, ACTION_DIM), jnp.float32),
        pltpu.SemaphoreType.DMA,
        pltpu.SemaphoreType.DMA,
        pltpu.SemaphoreType.DMA,
        pltpu.SemaphoreType.DMA,
        pltpu.SemaphoreType.DMA,
        pltpu.SemaphoreType.DMA,
        pltpu.SemaphoreType.DMA,
        pltpu.SemaphoreType.DMA,
    ],
    compiler_params=pltpu.CompilerParams(use_tc_tiling_on_sc=False),
)


def kernel(action, table):
    idx = action.reshape(NW * 2, K // 2, CHUNK).astype(jnp.int32)
    out = _gather(idx, table)
    return out.reshape(BATCH, HIST * ACTION_DIM)
